# baseline (speedup 1.0000x reference)
import functools

import jax
import jax.numpy as jnp
from jax import lax
from jax.experimental import pallas as pl
from jax.experimental.pallas import tpu as pltpu
from jax.experimental.pallas import tpu_sc as plsc

_ENTITY_NUM = 100000
_DIM = 64
_MARGIN = 1.0
_BATCH = 16384

_NC = 2
_NS = 16
_NW = _NC * _NS
_ROWS = _BATCH // _NW
_CH = 128
_NCHUNK = _ROWS // _CH
_NGRP = _DIM // 16


def _sc_partials(cur, cor, ent_emb, rel_emb):
    mesh = plsc.VectorSubcoreMesh(core_axis_name="c", subcore_axis_name="s")

    @functools.partial(
        pl.kernel,
        mesh=mesh,
        out_type=jax.ShapeDtypeStruct((2, _NW, _DIM), jnp.float32),
        scratch_types=[
            pltpu.VMEM((_CH, 3), jnp.int32),
            pltpu.VMEM((3, _CH), jnp.int32),
            pltpu.VMEM((_CH, _DIM), jnp.float32),
            pltpu.VMEM((_CH, _DIM), jnp.float32),
            pltpu.VMEM((_CH, _DIM), jnp.float32),
            pltpu.VMEM((_DIM,), jnp.float32),
            pltpu.SemaphoreType.DMA,
        ],
        compiler_params=pltpu.CompilerParams(
            use_tc_tiling_on_sc=False, needs_layout_passes=False),
    )
    def body(cur_hbm, cor_hbm, ent_hbm, rel_hbm, out_hbm,
             raw_v, idx_v, hv, rv, tv, out_v, sem):
        wid = lax.axis_index("s") * _NC + lax.axis_index("c")
        base = wid * _ROWS
        lane = lax.iota(jnp.int32, 16)

        for l, lst in ((0, cur_hbm), (1, cor_hbm)):
            acc = tuple(jnp.zeros((16,), jnp.float32) for _ in range(_NGRP))
            for c in range(_NCHUNK):
                pltpu.sync_copy(lst.at[pl.ds(base + c * _CH, _CH)], raw_v)
                for j in range(3):
                    colj = jnp.full((16,), j, jnp.int32)
                    for k in range(_CH // 16):
                        rows16 = lane + (16 * k)
                        idx_v[j, pl.ds(16 * k, 16)] = plsc.load_gather(
                            raw_v, [rows16, colj])
                g0 = pltpu.async_copy(ent_hbm.at[idx_v.at[0]], hv, sem)
                g1 = pltpu.async_copy(rel_hbm.at[idx_v.at[1]], rv, sem)
                g2 = pltpu.async_copy(ent_hbm.at[idx_v.at[2]], tv, sem)
                g0.wait()
                g1.wait()
                g2.wait()

                def row(rr, carry):
                    new = []
                    for g in range(_NGRP):
                        sl = pl.ds(g * 16, 16)
                        v = hv[rr, sl] + rv[rr, sl] - tv[rr, sl]
                        new.append(carry[g] + v * v)
                    return tuple(new)

                acc = lax.fori_loop(0, _CH, row, acc)

            for g in range(_NGRP):
                out_v[pl.ds(g * 16, 16)] = acc[g]
            pltpu.sync_copy(out_v, out_hbm.at[l, wid])

    return body(cur, cor, ent_emb, rel_emb)


def _finish(partials):
    def body(p_ref, o_ref):
        p = p_ref[...]
        s = jnp.sum(p, axis=1)
        d = jnp.sqrt(s)
        m = jnp.maximum(d[0:1] - d[1:2] + _MARGIN, 0.0)
        o_ref[...] = jnp.sum(m, axis=1, keepdims=True) * (1.0 / _DIM)

    return pl.pallas_call(
        body,
        out_shape=jax.ShapeDtypeStruct((1, 1), jnp.float32),
    )(partials)


@jax.jit
def kernel(current_list, corrupt_list, ent_emb, rel_emb):
    partials = _sc_partials(current_list, corrupt_list, ent_emb, rel_emb)
    loss = _finish(partials)
    return loss[0, 0]


# speedup vs baseline: 1.0049x; 1.0049x over previous
import functools

import jax
import jax.numpy as jnp
from jax import lax
from jax.experimental import pallas as pl
from jax.experimental.pallas import tpu as pltpu
from jax.experimental.pallas import tpu_sc as plsc

_ENTITY_NUM = 100000
_DIM = 64
_MARGIN = 1.0
_BATCH = 16384

_NC = 2
_NS = 16
_NW = _NC * _NS
_NPASS = _DIM // _NW
_H = _BATCH // 2
_CI = 2048


def _sc_partials(cur_t, cor_t, ent_t, rel_t):
    mesh = plsc.VectorSubcoreMesh(core_axis_name="c", subcore_axis_name="s")

    @functools.partial(
        pl.kernel,
        mesh=mesh,
        out_type=jax.ShapeDtypeStruct((2, _DIM, 128), jnp.float32),
        scratch_types=[
            pltpu.VMEM((1, _ENTITY_NUM), jnp.float32),
            pltpu.VMEM((2, _H), jnp.float32),
            pltpu.VMEM((4, _CI), jnp.int32),
            pltpu.VMEM((2, 1, 128), jnp.float32),
        ],
        compiler_params=pltpu.CompilerParams(
            use_tc_tiling_on_sc=True, needs_layout_passes=False),
    )
    def body(cur_hbm, cor_hbm, ent_hbm, rel_hbm, out_hbm,
             row_v, vsum, idxb, accv):
        wid = lax.axis_index("s") * _NC + lax.axis_index("c")
        z16 = jnp.zeros((16,), jnp.int32)

        for p in range(_NPASS):
            d = wid + _NW * p
            accs = [jnp.zeros((16,), jnp.float32) for _ in range(2)]
            for h2 in range(2):
                bb = h2 * _H
                pltpu.sync_copy(ent_hbm.at[pl.ds(d, 1)], row_v)
                for c in range(_H // _CI):
                    cb = bb + c * _CI
                    pltpu.sync_copy(cur_hbm.at[pl.ds(0, 1), pl.ds(cb, _CI)],
                                    idxb.at[pl.ds(0, 1)])
                    pltpu.sync_copy(cur_hbm.at[pl.ds(2, 1), pl.ds(cb, _CI)],
                                    idxb.at[pl.ds(1, 1)])
                    pltpu.sync_copy(cor_hbm.at[pl.ds(0, 1), pl.ds(cb, _CI)],
                                    idxb.at[pl.ds(2, 1)])
                    pltpu.sync_copy(cor_hbm.at[pl.ds(2, 1), pl.ds(cb, _CI)],
                                    idxb.at[pl.ds(3, 1)])

                    def eb(i, _, c=c):
                        off = i * 16
                        for l in range(2):
                            hi = idxb[2 * l, pl.ds(off, 16)]
                            ti = idxb[2 * l + 1, pl.ds(off, 16)]
                            gh = plsc.load_gather(row_v, [z16, hi])
                            gt = plsc.load_gather(row_v, [z16, ti])
                            vsum[l, pl.ds(c * _CI + off, 16)] = gh - gt
                        return 0

                    lax.fori_loop(0, _CI // 16, eb, 0)
                pltpu.sync_copy(rel_hbm.at[pl.ds(d, 1)], row_v)
                for c in range(_H // _CI):
                    cb = bb + c * _CI
                    pltpu.sync_copy(cur_hbm.at[pl.ds(1, 1), pl.ds(cb, _CI)],
                                    idxb.at[pl.ds(0, 1)])
                    pltpu.sync_copy(cor_hbm.at[pl.ds(1, 1), pl.ds(cb, _CI)],
                                    idxb.at[pl.ds(1, 1)])

                    def rb(i, carry, c=c):
                        off = i * 16
                        a0, a1 = carry
                        ri = idxb[0, pl.ds(off, 16)]
                        rci = idxb[1, pl.ds(off, 16)]
                        v0 = vsum[0, pl.ds(c * _CI + off, 16)] \
                            + plsc.load_gather(row_v, [z16, ri])
                        v1 = vsum[1, pl.ds(c * _CI + off, 16)] \
                            + plsc.load_gather(row_v, [z16, rci])
                        return (a0 + v0 * v0, a1 + v1 * v1)

                    accs = list(lax.fori_loop(0, _CI // 16, rb, tuple(accs)))
            for l in range(2):
                accv[l, 0, pl.ds(0, 16)] = accs[l]
                pltpu.sync_copy(accv.at[pl.ds(l, 1)],
                                out_hbm.at[pl.ds(l, 1), pl.ds(d, 1)])

    return body(cur_t, cor_t, ent_t, rel_t)


def _finish(partials):
    def body(p_ref, o_ref):
        p = p_ref[:, :, 0:16]
        s = jnp.sum(p, axis=2)
        dist = jnp.sqrt(s)
        m = jnp.maximum(dist[0:1] - dist[1:2] + _MARGIN, 0.0)
        o_ref[...] = jnp.sum(m, axis=1, keepdims=True) * (1.0 / _DIM)

    return pl.pallas_call(
        body,
        out_shape=jax.ShapeDtypeStruct((1, 1), jnp.float32),
    )(partials)


@jax.jit
def kernel(current_list, corrupt_list, ent_emb, rel_emb):
    partials = _sc_partials(
        current_list.T, corrupt_list.T, ent_emb.T, rel_emb.T)
    loss = _finish(partials)
    return loss[0, 0]


# speedup vs baseline: 1.4263x; 1.4194x over previous
import functools

import jax
import jax.numpy as jnp
from jax import lax
from jax.experimental import pallas as pl
from jax.experimental.pallas import tpu as pltpu
from jax.experimental.pallas import tpu_sc as plsc

_ENTITY_NUM = 100000
_DIM = 64
_MARGIN = 1.0
_BATCH = 16384

_NC = 2
_NS = 16
_NW = _NC * _NS
_NPASS = _DIM // _NW
_H = _BATCH // 2
_CI = 1024
_NCH = _H // _CI
_UNROLL = 4


def _sc_partials(cur_t, cor_t, ent_t, rel_t):
    mesh = plsc.VectorSubcoreMesh(core_axis_name="c", subcore_axis_name="s")

    @functools.partial(
        pl.kernel,
        mesh=mesh,
        out_type=jax.ShapeDtypeStruct((2, _DIM, 128), jnp.float32),
        scratch_types=[
            pltpu.VMEM((1, _ENTITY_NUM), jnp.float32),
            pltpu.VMEM((2, _H), jnp.float32),
            pltpu.VMEM((2, 4, _CI), jnp.int32),
            pltpu.VMEM((2, 1, 128), jnp.float32),
            pltpu.SemaphoreType.DMA,
        ],
        compiler_params=pltpu.CompilerParams(
            use_tc_tiling_on_sc=True, needs_layout_passes=False),
    )
    def body(cur_hbm, cor_hbm, ent_hbm, rel_hbm, out_hbm,
             row_v, vsum, idxb, accv, sem):
        wid = lax.axis_index("s") * _NC + lax.axis_index("c")
        z16 = jnp.zeros((16,), jnp.int32)

        def idx_pairs(cols, c, bb):
            ds = pl.ds(bb + c * _CI, _CI)
            buf = lax.rem(c, 2)
            return [(src.at[pl.ds(col, 1), ds], idxb.at[buf, pl.ds(j, 1)])
                    for j, (src, col) in enumerate(cols)]

        def start_idx(cols, c, bb):
            for s, t in idx_pairs(cols, c, bb):
                pltpu.async_copy(s, t, sem)

        def wait_idx(cols, c, bb):
            for s, t in idx_pairs(cols, c, bb):
                pltpu.make_async_copy(s, t, sem).wait()

        ecols = [(cur_hbm, 0), (cur_hbm, 2), (cor_hbm, 0), (cor_hbm, 2)]
        rcols = [(cur_hbm, 1), (cor_hbm, 1)]

        def pass_body(p, _):
            d = wid + _NW * p

            def half_body(h2, accs):
                bb = h2 * _H
                pltpu.sync_copy(ent_hbm.at[pl.ds(d, 1)], row_v)
                start_idx(ecols, 0, bb)

                def e_chunk(c, _):
                    wait_idx(ecols, c, bb)

                    @pl.when(c + 1 < _NCH)
                    def _():
                        start_idx(ecols, c + 1, bb)

                    buf = lax.rem(c, 2)

                    def eb(i, _):
                        for u in range(_UNROLL):
                            off = i * (16 * _UNROLL) + u * 16
                            so = c * _CI + off
                            for l in range(2):
                                hi = idxb[buf, 2 * l, pl.ds(off, 16)]
                                ti = idxb[buf, 2 * l + 1, pl.ds(off, 16)]
                                gh = plsc.load_gather(row_v, [z16, hi])
                                gt = plsc.load_gather(row_v, [z16, ti])
                                vsum[l, pl.ds(so, 16)] = gh - gt
                        return 0

                    return lax.fori_loop(0, _CI // (16 * _UNROLL), eb, 0)

                lax.fori_loop(0, _NCH, e_chunk, 0)

                pltpu.sync_copy(rel_hbm.at[pl.ds(d, 1)], row_v)
                start_idx(rcols, 0, bb)

                def r_chunk(c, accs):
                    wait_idx(rcols, c, bb)

                    @pl.when(c + 1 < _NCH)
                    def _():
                        start_idx(rcols, c + 1, bb)

                    buf = lax.rem(c, 2)

                    def rb(i, carry):
                        a0, a1 = carry
                        for u in range(_UNROLL):
                            off = i * (16 * _UNROLL) + u * 16
                            so = c * _CI + off
                            ri = idxb[buf, 0, pl.ds(off, 16)]
                            rci = idxb[buf, 1, pl.ds(off, 16)]
                            v0 = vsum[0, pl.ds(so, 16)] \
                                + plsc.load_gather(row_v, [z16, ri])
                            v1 = vsum[1, pl.ds(so, 16)] \
                                + plsc.load_gather(row_v, [z16, rci])
                            a0 = a0 + v0 * v0
                            a1 = a1 + v1 * v1
                        return (a0, a1)

                    return lax.fori_loop(0, _CI // (16 * _UNROLL), rb, accs)

                return lax.fori_loop(0, _NCH, r_chunk, accs)

            accs = lax.fori_loop(
                0, 2, half_body,
                (jnp.zeros((16,), jnp.float32), jnp.zeros((16,), jnp.float32)))
            for l in range(2):
                accv[l, 0, pl.ds(0, 16)] = accs[l]
                pltpu.sync_copy(accv.at[pl.ds(l, 1)],
                                out_hbm.at[pl.ds(l, 1), pl.ds(d, 1)])
            return 0

        lax.fori_loop(0, _NPASS, pass_body, 0)

    return body(cur_t, cor_t, ent_t, rel_t)


def _finish(partials):
    def body(p_ref, o_ref):
        p = p_ref[:, :, 0:16]
        s = jnp.sum(p, axis=2)
        dist = jnp.sqrt(s)
        m = jnp.maximum(dist[0:1] - dist[1:2] + _MARGIN, 0.0)
        o_ref[...] = jnp.sum(m, axis=1, keepdims=True) * (1.0 / _DIM)

    return pl.pallas_call(
        body,
        out_shape=jax.ShapeDtypeStruct((1, 1), jnp.float32),
    )(partials)


@jax.jit
def kernel(current_list, corrupt_list, ent_emb, rel_emb):
    partials = _sc_partials(
        current_list.T, corrupt_list.T, ent_emb.T, rel_emb.T)
    loss = _finish(partials)
    return loss[0, 0]


# speedup vs baseline: 1.6942x; 1.1878x over previous
import functools

import jax
import jax.numpy as jnp
from jax import lax
from jax.experimental import pallas as pl
from jax.experimental.pallas import tpu as pltpu
from jax.experimental.pallas import tpu_sc as plsc

_ENTITY_NUM = 100000
_DIM = 64
_MARGIN = 1.0
_BATCH = 16384

_NC = 2
_NS = 16
_NW = _NC * _NS
_NPASS = _DIM // _NW
_CI = 2048
_NCH = _BATCH // _CI
_UNROLL = 8


def _sc_partials(cur_t, cor_t, ent_t, rel_t):
    mesh = plsc.VectorSubcoreMesh(core_axis_name="c", subcore_axis_name="s")

    @functools.partial(
        pl.kernel,
        mesh=mesh,
        out_type=jax.ShapeDtypeStruct((2, _DIM, 128), jnp.float32),
        scratch_types=[
            pltpu.VMEM((1, _ENTITY_NUM), jnp.float32),
            pltpu.VMEM((2, _CI), jnp.float32),
            pltpu.VMEM((2, 4, _CI), jnp.int32),
            pltpu.VMEM((2, 1, 128), jnp.float32),
            pltpu.VMEM_SHARED((2, _BATCH), jnp.float32),
            pltpu.SemaphoreType.DMA,
        ],
        compiler_params=pltpu.CompilerParams(
            use_tc_tiling_on_sc=True, needs_layout_passes=False),
    )
    def body(cur_hbm, cor_hbm, ent_hbm, rel_hbm, out_hbm,
             row_v, vbuf, idxb, accv, vsp, sem):
        wid = lax.axis_index("s") * _NC + lax.axis_index("c")
        sid = lax.axis_index("s")
        z16 = jnp.zeros((16,), jnp.int32)

        def idx_pairs(cols, c):
            ds = pl.ds(c * _CI, _CI)
            buf = lax.rem(c, 2)
            return [(src.at[pl.ds(col, 1), ds], idxb.at[buf, pl.ds(j, 1)])
                    for j, (src, col) in enumerate(cols)]

        def start_idx(cols, c):
            for s, t in idx_pairs(cols, c):
                pltpu.async_copy(s, t, sem)

        def wait_idx(cols, c):
            for s, t in idx_pairs(cols, c):
                pltpu.make_async_copy(s, t, sem).wait()

        ecols = [(cur_hbm, 0), (cur_hbm, 2), (cor_hbm, 0), (cor_hbm, 2)]
        rcols = [(cur_hbm, 1), (cor_hbm, 1)]

        def pass_body(p, _):
            d = wid + _NW * p

            pltpu.sync_copy(ent_hbm.at[pl.ds(d, 1)], row_v)
            start_idx(ecols, 0)

            def e_chunk(c, _):
                wait_idx(ecols, c)

                @pl.when(c + 1 < _NCH)
                def _():
                    start_idx(ecols, c + 1)

                buf = lax.rem(c, 2)

                def eb(i, _):
                    for u in range(_UNROLL):
                        off = i * (16 * _UNROLL) + u * 16
                        for l in range(2):
                            hi = idxb[buf, 2 * l, pl.ds(off, 16)]
                            ti = idxb[buf, 2 * l + 1, pl.ds(off, 16)]
                            gh = plsc.load_gather(row_v, [z16, hi])
                            gt = plsc.load_gather(row_v, [z16, ti])
                            vbuf[l, pl.ds(off, 16)] = gh - gt
                    return 0

                lax.fori_loop(0, _CI // (16 * _UNROLL), eb, 0)
                for l in range(2):
                    pltpu.sync_copy(vbuf.at[pl.ds(l, 1)],
                                    vsp.at[pl.ds(l, 1),
                                           pl.ds(c * _CI, _CI)])
                return 0

            lax.fori_loop(0, _NCH, e_chunk, 0)

            pltpu.sync_copy(rel_hbm.at[pl.ds(d, 1)], row_v)
            start_idx(rcols, 0)

            def r_chunk(c, accs):
                wait_idx(rcols, c)

                @pl.when(c + 1 < _NCH)
                def _():
                    start_idx(rcols, c + 1)

                buf = lax.rem(c, 2)
                for l in range(2):
                    pltpu.sync_copy(vsp.at[pl.ds(l, 1),
                                           pl.ds(c * _CI, _CI)],
                                    vbuf.at[pl.ds(l, 1)])

                def rb(i, carry):
                    a0, a1 = carry
                    for u in range(_UNROLL):
                        off = i * (16 * _UNROLL) + u * 16
                        ri = idxb[buf, 0, pl.ds(off, 16)]
                        rci = idxb[buf, 1, pl.ds(off, 16)]
                        v0 = vbuf[0, pl.ds(off, 16)] \
                            + plsc.load_gather(row_v, [z16, ri])
                        v1 = vbuf[1, pl.ds(off, 16)] \
                            + plsc.load_gather(row_v, [z16, rci])
                        a0 = a0 + v0 * v0
                        a1 = a1 + v1 * v1
                    return (a0, a1)

                return lax.fori_loop(0, _CI // (16 * _UNROLL), rb, accs)

            accs = lax.fori_loop(
                0, _NCH, r_chunk,
                (jnp.zeros((16,), jnp.float32), jnp.zeros((16,), jnp.float32)))
            for l in range(2):
                accv[l, 0, pl.ds(0, 16)] = accs[l]
                pltpu.sync_copy(accv.at[pl.ds(l, 1)],
                                out_hbm.at[pl.ds(l, 1), pl.ds(d, 1)])
            return 0

        lax.fori_loop(0, _NPASS, pass_body, 0)

    return body(cur_t, cor_t, ent_t, rel_t)


def _finish(partials):
    def body(p_ref, o_ref):
        p = p_ref[:, :, 0:16]
        s = jnp.sum(p, axis=2)
        dist = jnp.sqrt(s)
        m = jnp.maximum(dist[0:1] - dist[1:2] + _MARGIN, 0.0)
        o_ref[...] = jnp.sum(m, axis=1, keepdims=True) * (1.0 / _DIM)

    return pl.pallas_call(
        body,
        out_shape=jax.ShapeDtypeStruct((1, 1), jnp.float32),
    )(partials)


@jax.jit
def kernel(current_list, corrupt_list, ent_emb, rel_emb):
    partials = _sc_partials(
        current_list.T, corrupt_list.T, ent_emb.T, rel_emb.T)
    loss = _finish(partials)
    return loss[0, 0]
